# Initial kernel scaffold; baseline (speedup 1.0000x reference)
import jax
import jax.numpy as jnp
from jax.experimental import pallas as pl


def kernel(x, edge_index, W1, b1, W2, b2, Wl, bl):
    raise NotImplementedError("write your pallas kernel here")


# speedup vs baseline: 12.0995x; 12.0995x over previous
import functools

import jax
import jax.numpy as jnp
from jax import lax
from jax.experimental import pallas as pl
from jax.experimental.pallas import tpu as pltpu
from jax.experimental.pallas import tpu_sc as plsc

N = 8192
E = 262144
H = 64
OUT = 64

NC = 2
NS = 16
NW = NC * NS
CHUNK = 128
EPW = E // NW
CHUNKS = EPW // CHUNK
ROWS_PER_TILE = N // NS
DEGW = 128
HP = 128

@functools.cache
def _sc_kernels():
  mesh = plsc.VectorSubcoreMesh(core_axis_name="c", subcore_axis_name="s")

  @functools.partial(
    pl.kernel,
    out_type=jax.ShapeDtypeStruct((NC, N, DEGW), jnp.float32),
    mesh=mesh,
    scratch_types=[
        pltpu.VMEM((CHUNKS, CHUNK), jnp.int32),
        pltpu.VMEM((CHUNK, DEGW), jnp.float32),
        pltpu.VMEM_SHARED((N, DEGW), jnp.float32),
    ],
  )
  def _sc_deg(dst_hbm, ones_hbm, zeros_hbm, out_hbm, dst_v, ones_v, acc_sh):
    cid = lax.axis_index("c")
    sid = lax.axis_index("s")
    wid = cid * NS + sid

    base = sid * ROWS_PER_TILE
    pltpu.sync_copy(zeros_hbm.at[pl.ds(base, ROWS_PER_TILE)],
                    acc_sh.at[pl.ds(base, ROWS_PER_TILE)])
    pltpu.sync_copy(ones_hbm, ones_v)
    pltpu.sync_copy(dst_hbm.at[wid], dst_v)
    plsc.subcore_barrier()

    def body(j, carry):
        pltpu.sync_copy(ones_v, acc_sh.at[dst_v.at[j]], add=True)
        return carry

    lax.fori_loop(0, CHUNKS, body, 0)
    plsc.subcore_barrier()
    pltpu.sync_copy(acc_sh.at[pl.ds(base, ROWS_PER_TILE)],
                    out_hbm.at[cid, pl.ds(base, ROWS_PER_TILE)])


  @functools.partial(
    pl.kernel,
    out_type=jax.ShapeDtypeStruct((NC, N, HP), jnp.float32),
    mesh=mesh,
    scratch_types=[
        pltpu.VMEM((CHUNKS, CHUNK), jnp.int32),
        pltpu.VMEM((CHUNKS, CHUNK), jnp.int32),
        pltpu.VMEM((CHUNK, HP), jnp.float32),
        pltpu.VMEM_SHARED((N, HP), jnp.float32),
        pltpu.SemaphoreType.DMA,
    ],
  )
  def _sc_scatter(p_hbm, src_hbm, dst_hbm, zeros_hbm, out_hbm,
                src_v, dst_v, rows_v, acc_sh, sem):
    cid = lax.axis_index("c")
    sid = lax.axis_index("s")
    wid = cid * NS + sid

    base = sid * ROWS_PER_TILE
    pltpu.sync_copy(zeros_hbm.at[pl.ds(base, ROWS_PER_TILE)],
                    acc_sh.at[pl.ds(base, ROWS_PER_TILE)])
    pltpu.sync_copy(src_hbm.at[wid], src_v)
    pltpu.sync_copy(dst_hbm.at[wid], dst_v)
    plsc.subcore_barrier()

    def body(j, carry):
        pltpu.async_copy(p_hbm.at[src_v.at[j]], rows_v, sem).wait()
        pltpu.sync_copy(rows_v, acc_sh.at[dst_v.at[j]], add=True)
        return carry

    lax.fori_loop(0, CHUNKS, body, 0)
    plsc.subcore_barrier()
    pltpu.sync_copy(acc_sh.at[pl.ds(base, ROWS_PER_TILE)],
                    out_hbm.at[cid, pl.ds(base, ROWS_PER_TILE)])

  return _sc_deg, _sc_scatter


def _mm_body(x_ref, w_ref, degt_ref, out_ref):
    h = jnp.dot(x_ref[...], w_ref[...], preferred_element_type=jnp.float32)
    deg = degt_ref[:, 0:1] + degt_ref[:, 1:2] + 1.0
    out_ref[...] = jnp.concatenate([h * lax.rsqrt(deg), jnp.zeros_like(h)],
                                   axis=1)


def _mm_xw1(x, w1, degt):
    br = 512
    return pl.pallas_call(
        _mm_body,
        grid=(N // br,),
        in_specs=[
            pl.BlockSpec((br, N), lambda i: (i, 0)),
            pl.BlockSpec((N, H), lambda i: (0, 0)),
            pl.BlockSpec((br, 2), lambda i: (i, 0)),
        ],
        out_specs=pl.BlockSpec((br, HP), lambda i: (i, 0)),
        out_shape=jax.ShapeDtypeStruct((N, HP), jnp.float32),
    )(x, w1, degt)


def _mid_body(s_ref, p1_ref, degt_ref, w2_ref, b1_ref, out_ref):
    deg = degt_ref[:, 0:1] + degt_ref[:, 1:2] + 1.0
    dinv = lax.rsqrt(deg)
    s = s_ref[...]
    pre = dinv * (s[0, :, :H] + s[1, :, :H] + p1_ref[:, :H]) + b1_ref[...]
    h1 = jnp.maximum(pre, 0.0)
    h2 = jnp.dot(h1, w2_ref[...], preferred_element_type=jnp.float32)
    out_ref[...] = jnp.concatenate([dinv * h2, jnp.zeros_like(h2)], axis=1)


def _mid(s1, p1, degt, w2, b1row):
    br = 1024
    return pl.pallas_call(
        _mid_body,
        grid=(N // br,),
        in_specs=[
            pl.BlockSpec((NC, br, HP), lambda i: (0, i, 0)),
            pl.BlockSpec((br, HP), lambda i: (i, 0)),
            pl.BlockSpec((br, 2), lambda i: (i, 0)),
            pl.BlockSpec((H, H), lambda i: (0, 0)),
            pl.BlockSpec((1, H), lambda i: (0, 0)),
        ],
        out_specs=pl.BlockSpec((br, HP), lambda i: (i, 0)),
        out_shape=jax.ShapeDtypeStruct((N, HP), jnp.float32),
    )(s1, p1, degt, w2, b1row)


def _fin_body(wl3_ref, s_ref, p2_ref, degt_ref, b2_ref, bl_ref,
              out_ref, acc_ref):
    i = pl.program_id(0)

    @pl.when(i == 0)
    def _init():
        acc_ref[...] = jnp.zeros_like(acc_ref)

    deg = degt_ref[:, 0:1] + degt_ref[:, 1:2] + 1.0
    dinv = lax.rsqrt(deg)
    s = s_ref[...]
    h2 = dinv * (s[0, :, :H] + s[1, :, :H] + p2_ref[:, :H]) + b2_ref[...]
    y = wl3_ref[...] * h2[None, :, :]
    acc_ref[...] += jnp.sum(y, axis=1)

    @pl.when(i == pl.num_programs(0) - 1)
    def _done():
        out_ref[...] = (jnp.sum(acc_ref[...], axis=1, keepdims=True)
                        + bl_ref[...])


def _fin(wl3, s2, p2, degt, b2row, blcol):
    bn = 512
    return pl.pallas_call(
        _fin_body,
        grid=(N // bn,),
        in_specs=[
            pl.BlockSpec((OUT, bn, H), lambda i: (0, i, 0)),
            pl.BlockSpec((NC, bn, HP), lambda i: (0, i, 0)),
            pl.BlockSpec((bn, HP), lambda i: (i, 0)),
            pl.BlockSpec((bn, 2), lambda i: (i, 0)),
            pl.BlockSpec((1, H), lambda i: (0, 0)),
            pl.BlockSpec((OUT, 1), lambda i: (0, 0)),
        ],
        out_specs=pl.BlockSpec((OUT, 1), lambda i: (0, 0)),
        out_shape=jax.ShapeDtypeStruct((OUT, 1), jnp.float32),
        scratch_shapes=[pltpu.VMEM((OUT, H), jnp.float32)],
    )(wl3, s2, p2, degt, b2row, blcol)


def kernel(x, edge_index, W1, b1, W2, b2, Wl, bl):
    src3 = edge_index[0].reshape(NW, CHUNKS, CHUNK)
    dst3 = edge_index[1].reshape(NW, CHUNKS, CHUNK)
    ones_deg = jnp.ones((CHUNK, DEGW), jnp.float32)
    zeros_deg = jnp.zeros((N, DEGW), jnp.float32)
    zeros_nh = jnp.zeros((N, HP), jnp.float32)

    _sc_deg, _sc_scatter = _sc_kernels()
    degp = _sc_deg(dst3, ones_deg, zeros_deg)
    degt = jnp.concatenate([degp[0, :, 0:1], degp[1, :, 0:1]], axis=1)

    p1 = _mm_xw1(x, W1, degt)
    s1 = _sc_scatter(p1, src3, dst3, zeros_nh)
    p2 = _mid(s1, p1, degt, W2, b1.reshape(1, H))
    s2 = _sc_scatter(p2, src3, dst3, zeros_nh)
    out2d = _fin(Wl.reshape(OUT, N, H), s2, p2, degt,
                 b2.reshape(1, H), bl.reshape(OUT, 1))
    return out2d.reshape(OUT)


# speedup vs baseline: 23.6428x; 1.9540x over previous
import functools

import jax
import jax.numpy as jnp
from jax import lax
from jax.experimental import pallas as pl
from jax.experimental.pallas import tpu as pltpu
from jax.experimental.pallas import tpu_sc as plsc

N = 8192
E = 262144
H = 64
OUT = 64

NC = 2
NS = 16
NW = NC * NS
CHUNK = 128
EPW = E // NW
CHUNKS = EPW // CHUNK
ROWS_PER_TILE = N // NS
DEGW = 128
HP = 128

@functools.cache
def _sc_kernels():
  mesh = plsc.VectorSubcoreMesh(core_axis_name="c", subcore_axis_name="s")

  @functools.partial(
    pl.kernel,
    out_type=jax.ShapeDtypeStruct((NC, N, DEGW), jnp.float32),
    mesh=mesh,
    scratch_types=[
        pltpu.VMEM((CHUNKS, CHUNK), jnp.int32),
        pltpu.VMEM((CHUNK, DEGW), jnp.float32),
        pltpu.VMEM_SHARED((N, DEGW), jnp.float32),
        pltpu.SemaphoreType.DMA,
    ],
  )
  def _sc_deg(dst_hbm, ones_hbm, zeros_hbm, out_hbm, dst_v, ones_v, acc_sh,
              sem):
    cid = lax.axis_index("c")
    sid = lax.axis_index("s")
    wid = cid * NS + sid

    base = sid * ROWS_PER_TILE
    pltpu.sync_copy(zeros_hbm.at[pl.ds(base, ROWS_PER_TILE)],
                    acc_sh.at[pl.ds(base, ROWS_PER_TILE)])
    pltpu.sync_copy(ones_hbm, ones_v)
    pltpu.sync_copy(dst_hbm.at[wid], dst_v)
    plsc.subcore_barrier()

    def body(j, carry):
        pltpu.async_copy(ones_v, acc_sh.at[dst_v.at[j]], sem, add=True)
        return carry

    lax.fori_loop(0, CHUNKS, body, 0)

    def drain(j, carry):
        pltpu.make_async_copy(ones_v, acc_sh.at[dst_v.at[j]], sem).wait()
        return carry

    lax.fori_loop(0, CHUNKS, drain, 0)
    plsc.subcore_barrier()
    pltpu.sync_copy(acc_sh.at[pl.ds(base, ROWS_PER_TILE)],
                    out_hbm.at[cid, pl.ds(base, ROWS_PER_TILE)])


  nbuf = 2
  assert CHUNKS % nbuf == 0

  @functools.partial(
    pl.kernel,
    out_type=jax.ShapeDtypeStruct((NC, N, HP), jnp.float32),
    mesh=mesh,
    scratch_types=[
        pltpu.VMEM((CHUNKS, CHUNK), jnp.int32),
        pltpu.VMEM((CHUNKS, CHUNK), jnp.int32),
    ]
    + [pltpu.VMEM((CHUNK, HP), jnp.float32) for _ in range(nbuf)]
    + [pltpu.VMEM_SHARED((N, HP), jnp.float32)]
    + [pltpu.SemaphoreType.DMA for _ in range(2 * nbuf)],
  )
  def _sc_scatter(p_hbm, src_hbm, dst_hbm, zeros_hbm, out_hbm,
                  src_v, dst_v, *rest):
    rows = rest[:nbuf]
    acc_sh = rest[nbuf]
    gsem = rest[nbuf + 1:nbuf + 1 + nbuf]
    ssem = rest[nbuf + 1 + nbuf:]
    cid = lax.axis_index("c")
    sid = lax.axis_index("s")
    wid = cid * NS + sid

    base = sid * ROWS_PER_TILE
    pltpu.sync_copy(zeros_hbm.at[pl.ds(base, ROWS_PER_TILE)],
                    acc_sh.at[pl.ds(base, ROWS_PER_TILE)])
    pltpu.sync_copy(src_hbm.at[wid], src_v)
    pltpu.sync_copy(dst_hbm.at[wid], dst_v)
    plsc.subcore_barrier()

    for b in range(nbuf):
        pltpu.async_copy(p_hbm.at[src_v.at[b]], rows[b], gsem[b])

    def round_(g0, carry):
        for b in range(nbuf):
            j = g0 * nbuf + b
            pltpu.make_async_copy(p_hbm.at[src_v.at[j]], rows[b],
                                  gsem[b]).wait()
            pltpu.async_copy(rows[b], acc_sh.at[dst_v.at[j]], ssem[b],
                             add=True)

            @pl.when(j + nbuf < CHUNKS)
            def _prefetch():
                pltpu.make_async_copy(rows[b], acc_sh.at[dst_v.at[j]],
                                      ssem[b]).wait()
                pltpu.async_copy(p_hbm.at[src_v.at[j + nbuf]], rows[b],
                                 gsem[b])
        return carry

    lax.fori_loop(0, CHUNKS // nbuf, round_, 0)
    for b in range(nbuf):
        j = CHUNKS - nbuf + b
        pltpu.make_async_copy(rows[b], acc_sh.at[dst_v.at[j]],
                              ssem[b]).wait()
    plsc.subcore_barrier()
    pltpu.sync_copy(acc_sh.at[pl.ds(base, ROWS_PER_TILE)],
                    out_hbm.at[cid, pl.ds(base, ROWS_PER_TILE)])

  return _sc_deg, _sc_scatter


def _mm_body(x_ref, w_ref, out_ref):
    h = jnp.dot(x_ref[...], w_ref[...], preferred_element_type=jnp.float32)
    out_ref[...] = jnp.concatenate([h, jnp.zeros_like(h)], axis=1)


def _mm_xw1(x, w1):
    br = 512
    return pl.pallas_call(
        _mm_body,
        grid=(N // br,),
        in_specs=[
            pl.BlockSpec((br, N), lambda i: (i, 0)),
            pl.BlockSpec((N, H), lambda i: (0, 0)),
        ],
        out_specs=pl.BlockSpec((br, HP), lambda i: (i, 0)),
        out_shape=jax.ShapeDtypeStruct((N, HP), jnp.float32),
    )(x, w1)


def _p1_body(h_ref, degt_ref, out_ref):
    deg = degt_ref[:, 0:1] + degt_ref[:, 1:2] + 1.0
    hv = h_ref[:, :H]
    out_ref[...] = jnp.concatenate([hv * lax.rsqrt(deg), jnp.zeros_like(hv)],
                                   axis=1)


def _p1(hpad, degt):
    br = 1024
    return pl.pallas_call(
        _p1_body,
        grid=(N // br,),
        in_specs=[
            pl.BlockSpec((br, HP), lambda i: (i, 0)),
            pl.BlockSpec((br, 2), lambda i: (i, 0)),
        ],
        out_specs=pl.BlockSpec((br, HP), lambda i: (i, 0)),
        out_shape=jax.ShapeDtypeStruct((N, HP), jnp.float32),
    )(hpad, degt)


def _mid_body(s_ref, p1_ref, degt_ref, w2_ref, b1_ref, out_ref):
    deg = degt_ref[:, 0:1] + degt_ref[:, 1:2] + 1.0
    dinv = lax.rsqrt(deg)
    s = s_ref[...]
    pre = dinv * (s[0, :, :H] + s[1, :, :H] + p1_ref[:, :H]) + b1_ref[...]
    h1 = jnp.maximum(pre, 0.0)
    h2 = jnp.dot(h1, w2_ref[...], preferred_element_type=jnp.float32)
    out_ref[...] = jnp.concatenate([dinv * h2, jnp.zeros_like(h2)], axis=1)


def _mid(s1, p1, degt, w2, b1row):
    br = 1024
    return pl.pallas_call(
        _mid_body,
        grid=(N // br,),
        in_specs=[
            pl.BlockSpec((NC, br, HP), lambda i: (0, i, 0)),
            pl.BlockSpec((br, HP), lambda i: (i, 0)),
            pl.BlockSpec((br, 2), lambda i: (i, 0)),
            pl.BlockSpec((H, H), lambda i: (0, 0)),
            pl.BlockSpec((1, H), lambda i: (0, 0)),
        ],
        out_specs=pl.BlockSpec((br, HP), lambda i: (i, 0)),
        out_shape=jax.ShapeDtypeStruct((N, HP), jnp.float32),
    )(s1, p1, degt, w2, b1row)


def _h2_body(s_ref, p2_ref, degt_ref, b2_ref, out_ref):
    deg = degt_ref[:, 0:1] + degt_ref[:, 1:2] + 1.0
    dinv = lax.rsqrt(deg)
    s = s_ref[...]
    out_ref[...] = (dinv * (s[0, :, :H] + s[1, :, :H] + p2_ref[:, :H])
                    + b2_ref[...])


def _h2(s2, p2, degt, b2row):
    br = 1024
    return pl.pallas_call(
        _h2_body,
        grid=(N // br,),
        in_specs=[
            pl.BlockSpec((NC, br, HP), lambda i: (0, i, 0)),
            pl.BlockSpec((br, HP), lambda i: (i, 0)),
            pl.BlockSpec((br, 2), lambda i: (i, 0)),
            pl.BlockSpec((1, H), lambda i: (0, 0)),
        ],
        out_specs=pl.BlockSpec((br, H), lambda i: (i, 0)),
        out_shape=jax.ShapeDtypeStruct((N, H), jnp.float32),
    )(s2, p2, degt, b2row)


def _fin_body(wl_ref, flat_ref, bl_ref, out_ref, acc_ref):
    i = pl.program_id(0)

    @pl.when(i == 0)
    def _init():
        acc_ref[...] = jnp.zeros_like(acc_ref)

    acc_ref[...] += jnp.sum(wl_ref[...] * flat_ref[...], axis=1,
                            keepdims=True)

    @pl.when(i == pl.num_programs(0) - 1)
    def _done():
        out_ref[...] = acc_ref[...] + bl_ref[...]


def _fin(wl, flat, blcol):
    bk = 32768
    kblocks = (N * H) // bk
    return pl.pallas_call(
        _fin_body,
        grid=(kblocks,),
        in_specs=[
            pl.BlockSpec((OUT, bk), lambda i: (0, i)),
            pl.BlockSpec((1, bk), lambda i: (0, i)),
            pl.BlockSpec((OUT, 1), lambda i: (0, 0)),
        ],
        out_specs=pl.BlockSpec((OUT, 1), lambda i: (0, 0)),
        out_shape=jax.ShapeDtypeStruct((OUT, 1), jnp.float32),
        scratch_shapes=[pltpu.VMEM((OUT, 1), jnp.float32)],
    )(wl, flat, blcol)


def kernel(x, edge_index, W1, b1, W2, b2, Wl, bl):
    src3 = edge_index[0].reshape(NW, CHUNKS, CHUNK)
    dst3 = edge_index[1].reshape(NW, CHUNKS, CHUNK)
    ones_deg = jnp.ones((CHUNK, DEGW), jnp.float32)
    zeros_deg = jnp.zeros((N, DEGW), jnp.float32)
    zeros_nh = jnp.zeros((N, HP), jnp.float32)

    _sc_deg, _sc_scatter = _sc_kernels()
    degp = _sc_deg(dst3, ones_deg, zeros_deg)
    degt = jnp.concatenate([degp[0, :, 0:1], degp[1, :, 0:1]], axis=1)

    hpad = _mm_xw1(x, W1)
    p1 = _p1(hpad, degt)
    s1 = _sc_scatter(p1, src3, dst3, zeros_nh)
    p2 = _mid(s1, p1, degt, W2, b1.reshape(1, H))
    s2 = _sc_scatter(p2, src3, dst3, zeros_nh)
    h2 = _h2(s2, p2, degt, b2.reshape(1, H))
    out2d = _fin(Wl, h2.reshape(1, N * H), bl.reshape(OUT, 1))
    return out2d.reshape(OUT)
